# baseline (speedup 1.0000x reference)
import jax
import jax.numpy as jnp
from jax.experimental import pallas as pl

_N = 1_000_000
_D = 64
_K = 16
_BLK = 25_000
_GRID = _N // _BLK
_SEG = 1000
_NSEG = _BLK // _SEG


def _knn_kernel(q_ref, x_ref, od_ref, oi_ref):
    i = pl.program_id(0)

    @pl.when(i == 0)
    def _init():
        od_ref[...] = jnp.full((1, _K), jnp.inf, jnp.float32)
        oi_ref[...] = jnp.zeros((1, _K), jnp.int32)

    q = q_ref[...]
    x = x_ref[...]
    qn2 = jnp.sum(q * q)
    dot = jax.lax.dot_general(
        q, x, (((1,), (1,)), ((), ())),
        precision=jax.lax.Precision.HIGHEST,
        preferred_element_type=jnp.float32)
    ones = jnp.ones((1, _D), jnp.float32)
    n2 = jax.lax.dot_general(
        ones, x * x, (((1,), (1,)), ((), ())),
        precision=jax.lax.Precision.HIGHEST,
        preferred_element_type=jnp.float32)
    denom = jnp.maximum(jnp.sqrt(qn2) * jnp.sqrt(n2), 1e-8)
    d = 1.0 - dot / denom

    col = jax.lax.broadcasted_iota(jnp.int32, (1, _K), 1)
    segcol = jax.lax.broadcasted_iota(jnp.int32, (1, _SEG), 1)
    for s in range(_NSEG):
        seg = jax.lax.slice(d, (0, s * _SEG), (1, (s + 1) * _SEG))
        sm = jnp.min(seg)
        t = jnp.max(od_ref[...])

        @pl.when(sm < t)
        def _merge(seg=seg, s=s):
            base = i * _BLK + s * _SEG
            cd = jnp.concatenate([seg, od_ref[...]], axis=1)
            ci = jnp.concatenate([base + segcol, oi_ref[...]], axis=1)
            nd = jnp.zeros((1, _K), jnp.float32)
            ni = jnp.zeros((1, _K), jnp.int32)
            for j in range(_K):
                m = jnp.min(cd)
                am = jnp.min(jnp.where(cd == m, ci, jnp.int32(2147483647)))
                nd = jnp.where(col == j, m, nd)
                ni = jnp.where(col == j, am, ni)
                cd = jnp.where(ci == am, jnp.float32(jnp.inf), cd)
            od_ref[...] = nd
            oi_ref[...] = ni


def kernel(query, database):
    d, i = pl.pallas_call(
        _knn_kernel,
        grid=(_GRID,),
        in_specs=[
            pl.BlockSpec((1, _D), lambda i: (0, 0)),
            pl.BlockSpec((_BLK, _D), lambda i: (i, 0)),
        ],
        out_specs=[
            pl.BlockSpec((1, _K), lambda i: (0, 0)),
            pl.BlockSpec((1, _K), lambda i: (0, 0)),
        ],
        out_shape=[
            jax.ShapeDtypeStruct((1, _K), jnp.float32),
            jax.ShapeDtypeStruct((1, _K), jnp.int32),
        ],
    )(query, database)
    return (d, i)


# speedup vs baseline: 1.9272x; 1.9272x over previous
import jax
import jax.numpy as jnp
from jax.experimental import pallas as pl

_N = 1_000_000
_D = 64
_K = 16
_BLK = 25_000
_GRID = _N // _BLK


def _knn_kernel(q_ref, x_ref, od_ref, oi_ref):
    i = pl.program_id(0)

    @pl.when(i == 0)
    def _init():
        od_ref[...] = jnp.full((1, _K), jnp.inf, jnp.float32)
        oi_ref[...] = jnp.zeros((1, _K), jnp.int32)

    q = q_ref[...]
    x = x_ref[...]
    qn2 = jnp.sum(q * q)

    def _dg(l, r):
        return jax.lax.dot_general(
            l, r, (((1,), (1,)), ((), ())),
            preferred_element_type=jnp.float32)

    xh = x.astype(jnp.bfloat16)
    xl = (x - xh.astype(jnp.float32)).astype(jnp.bfloat16)
    qh = q.astype(jnp.bfloat16)
    ql = (q - qh.astype(jnp.float32)).astype(jnp.bfloat16)
    q2 = jnp.concatenate([qh, ql], axis=0)
    dh = _dg(q2, xh)
    dl = _dg(q2, xl)
    dot = (dh[0:1, :] + dh[1:2, :]) + (dl[0:1, :] + dl[1:2, :])
    s = x * x
    sh = s.astype(jnp.bfloat16)
    sl = (s - sh.astype(jnp.float32)).astype(jnp.bfloat16)
    ones = jnp.ones((1, _D), jnp.bfloat16)
    n2 = _dg(ones, sh) + _dg(ones, sl)
    denom = jnp.maximum(jnp.sqrt(qn2) * jnp.sqrt(n2), 1e-8)
    d = 1.0 - dot / denom

    bm = jnp.min(d)
    t = jnp.max(od_ref[...])

    @pl.when(bm < t)
    def _merge():
        idx = i * _BLK + jax.lax.broadcasted_iota(jnp.int32, (1, _BLK), 1)
        cd = jnp.concatenate([d, od_ref[...]], axis=1)
        ci = jnp.concatenate([idx, oi_ref[...]], axis=1)
        col = jax.lax.broadcasted_iota(jnp.int32, (1, _K), 1)
        nd = jnp.zeros((1, _K), jnp.float32)
        ni = jnp.zeros((1, _K), jnp.int32)
        for j in range(_K):
            m = jnp.min(cd)
            am = jnp.min(jnp.where(cd == m, ci, jnp.int32(2147483647)))
            nd = jnp.where(col == j, m, nd)
            ni = jnp.where(col == j, am, ni)
            cd = jnp.where(ci == am, jnp.float32(jnp.inf), cd)
        od_ref[...] = nd
        oi_ref[...] = ni


def kernel(query, database):
    d, i = pl.pallas_call(
        _knn_kernel,
        grid=(_GRID,),
        in_specs=[
            pl.BlockSpec((1, _D), lambda i: (0, 0)),
            pl.BlockSpec((_BLK, _D), lambda i: (i, 0)),
        ],
        out_specs=[
            pl.BlockSpec((1, _K), lambda i: (0, 0)),
            pl.BlockSpec((1, _K), lambda i: (0, 0)),
        ],
        out_shape=[
            jax.ShapeDtypeStruct((1, _K), jnp.float32),
            jax.ShapeDtypeStruct((1, _K), jnp.int32),
        ],
    )(query, database)
    return (d, i)
